# baseline (speedup 1.0000x reference)
import functools

import jax
import jax.numpy as jnp
from jax.experimental import pallas as pl
from jax.experimental.pallas import tpu as pltpu

_DILATIONS = (1, 2, 4)
_OFFS = (0, 1, 2, 4)
_P = 64


def _pattern_kernel(x_ref, w_ref, b_ref, m_ref, out_ref, *, n_rows):
    tb = x_ref.shape[0] // n_rows
    w3 = w_ref.shape[1]
    y = jnp.dot(x_ref[...], w_ref[...], preferred_element_type=jnp.float32)
    y = y.reshape(tb, n_rows, w3)
    s = jnp.maximum(y + b_ref[...].reshape(1, 1, w3), 0.0)
    r = jnp.sum(s * m_ref[...].reshape(1, n_rows, w3), axis=1)
    out_ref[...] = r[:, 0:_P] + r[:, _P:2 * _P] + r[:, 2 * _P:3 * _P]


@functools.partial(jax.jit, static_argnames=("block_boards",))
def _pattern_module(discarded_idx, card_features, W1, b1, W2, b2, W4, b4,
                    *, block_boards=256):
    B, N = discarded_idx.shape
    C = card_features.shape[-1]
    P = _P
    nb = len(_DILATIONS)
    K = len(_OFFS) * C

    slot_of = {d: _OFFS.index(d) for d in _DILATIONS}
    w_all = jnp.zeros((K, nb * P), jnp.float32)
    params = {1: (W1, b1), 2: (W2, b2), 4: (W4, b4)}
    b_cols = []
    for j, d in enumerate(_DILATIONS):
        Wd, bd = params[d]
        w_all = w_all.at[0:C, j * P:(j + 1) * P].set(jnp.transpose(Wd[:, :, 0]))
        so = slot_of[d] * C
        w_all = w_all.at[so:so + C, j * P:(j + 1) * P].set(
            jnp.transpose(Wd[:, :, 1]))
        b_cols.append(jnp.broadcast_to(bd.reshape(1, P), (1, P)))
    w_all = w_all.astype(jnp.bfloat16)
    b_all = jnp.concatenate(b_cols, axis=1).astype(jnp.float32)

    t = jnp.arange(N, dtype=jnp.int32).reshape(N, 1)
    m_cols = []
    for d in _DILATIONS:
        L = N - d
        m_cols.append(jnp.broadcast_to(
            (t < L).astype(jnp.float32) * (1.0 / L), (N, P)))
    m_all = jnp.concatenate(m_cols, axis=1)

    idx = discarded_idx.astype(jnp.int32)
    cols = []
    for off in _OFFS:
        if off == 0:
            cols.append(idx)
        else:
            cols.append(jnp.concatenate(
                [idx[:, off:], jnp.broadcast_to(idx[:, -1:], (B, off))],
                axis=1))
    idx4 = jnp.stack(cols, axis=-1)
    cf = card_features.astype(jnp.bfloat16)
    x = cf[idx4].reshape(B * N, K)

    tb = block_boards
    grid = B // tb
    body = functools.partial(_pattern_kernel, n_rows=N)
    out = pl.pallas_call(
        body,
        out_shape=jax.ShapeDtypeStruct((B, P), jnp.float32),
        grid=(grid,),
        in_specs=[
            pl.BlockSpec((tb * N, K), lambda i: (i, 0)),
            pl.BlockSpec((K, nb * P), lambda i: (0, 0)),
            pl.BlockSpec((1, nb * P), lambda i: (0, 0)),
            pl.BlockSpec((N, nb * P), lambda i: (0, 0)),
        ],
        out_specs=pl.BlockSpec((tb, P), lambda i: (i, 0)),
        compiler_params=pltpu.CompilerParams(
            dimension_semantics=("parallel",)),
    )(x, w_all, b_all, m_all)
    return out


def kernel(discarded_idx, card_features, W1, b1, W2, b2, W4, b4):
    return _pattern_module(discarded_idx, card_features,
                           W1, b1, W2, b2, W4, b4)


# speedup vs baseline: 15.5608x; 15.5608x over previous
import functools

import jax
import jax.numpy as jnp
from jax.experimental import pallas as pl
from jax.experimental.pallas import tpu as pltpu

_P = 64
_NCARDS = 40


def _pattern_kernel(idx_ref, w_ref, b_ref, m_ref, out_ref, *, n_rows):
    rows = idx_ref.shape[0]
    tb = rows // n_rows
    nc = _NCARDS
    idx3 = idx_ref[...]
    iota = jax.lax.broadcasted_iota(jnp.int32, (rows, nc), 1)
    oh = jnp.concatenate(
        [(idx3[:, s:s + 1] == iota).astype(jnp.bfloat16) for s in range(3)],
        axis=1)
    y = jnp.dot(oh, w_ref[...], preferred_element_type=jnp.float32)
    y = (y + b_ref[...].reshape(1, 256)).reshape(tb, n_rows, 256)
    b4s = jnp.concatenate(
        [y[:, 4:, 192:256], jnp.zeros((tb, 4, _P), jnp.float32)], axis=1)
    act4 = y[:, :, 128:192] + b4s
    s12 = jnp.maximum(y[:, :, 0:128], 0.0)
    s4 = jnp.maximum(act4, 0.0)
    m = m_ref[...].reshape(1, n_rows, 192)
    r12 = jnp.sum(s12 * m[:, :, 0:128], axis=1)
    r4 = jnp.sum(s4 * m[:, :, 128:192], axis=1)
    out_ref[...] = r12[:, 0:_P] + r12[:, _P:2 * _P] + r4


@functools.partial(jax.jit, static_argnames=("block_boards",))
def _pattern_module(discarded_idx, card_features, W1, b1, W2, b2, W4, b4,
                    *, block_boards=128):
    B, N = discarded_idx.shape
    P = _P
    nc = _NCARDS

    def tap(Wd, k):
        return jnp.dot(card_features, jnp.transpose(Wd[:, :, k]))

    w = jnp.zeros((3 * nc, 4 * P), jnp.float32)
    w = w.at[0:nc, 0 * P:1 * P].set(tap(W1, 0))
    w = w.at[nc:2 * nc, 0 * P:1 * P].set(tap(W1, 1))
    w = w.at[0:nc, 1 * P:2 * P].set(tap(W2, 0))
    w = w.at[2 * nc:3 * nc, 1 * P:2 * P].set(tap(W2, 1))
    w = w.at[0:nc, 2 * P:3 * P].set(tap(W4, 0))
    w = w.at[0:nc, 3 * P:4 * P].set(tap(W4, 1))
    w = w.astype(jnp.bfloat16)
    bias = jnp.concatenate(
        [b1.reshape(1, P), b2.reshape(1, P), b4.reshape(1, P),
         jnp.zeros((1, P), jnp.float32)], axis=1)

    t = jnp.arange(N, dtype=jnp.int32).reshape(N, 1)
    m_cols = []
    for d in (1, 2, 4):
        L = N - d
        m_cols.append(jnp.broadcast_to(
            (t < L).astype(jnp.float32) * (1.0 / L), (N, P)))
    m_all = jnp.concatenate(m_cols, axis=1)

    idx = discarded_idx.astype(jnp.int32)
    cols = [idx]
    for off in (1, 2):
        cols.append(jnp.concatenate(
            [idx[:, off:], jnp.broadcast_to(idx[:, -1:], (B, off))], axis=1))
    idx3 = jnp.stack(cols, axis=-1).reshape(B * N, 3)

    tb = block_boards
    body = functools.partial(_pattern_kernel, n_rows=N)
    out = pl.pallas_call(
        body,
        out_shape=jax.ShapeDtypeStruct((B, P), jnp.float32),
        grid=(B // tb,),
        in_specs=[
            pl.BlockSpec((tb * N, 3), lambda i: (i, 0)),
            pl.BlockSpec((3 * nc, 4 * P), lambda i: (0, 0)),
            pl.BlockSpec((1, 4 * P), lambda i: (0, 0)),
            pl.BlockSpec((N, 3 * P), lambda i: (0, 0)),
        ],
        out_specs=pl.BlockSpec((tb, P), lambda i: (i, 0)),
        compiler_params=pltpu.CompilerParams(
            dimension_semantics=("parallel",)),
    )(idx3, w, bias, m_all)
    return out


def kernel(discarded_idx, card_features, W1, b1, W2, b2, W4, b4):
    return _pattern_module(discarded_idx, card_features,
                           W1, b1, W2, b2, W4, b4)
